# baseline (speedup 1.0000x reference)
import functools

import jax
import jax.numpy as jnp
from jax import lax
from jax.experimental import pallas as pl
from jax.experimental.pallas import tpu as pltpu
from jax.experimental.pallas import tpu_sc as plsc
from jax.scipy.linalg import block_diag as _block_diag

N = 10000
E = 320000
IN_DIM = 128
HID = 32
NC_OUT = 16

NCORES = 2
NSUB = 16
NW = NCORES * NSUB
EPT = E // NW
BATCH = 125
NB = EPT // BATCH
NPAD = 10240
NODES_PER_SUB = NPAD // NSUB
NSLOT = 8
DEG_GRP = 10

N4 = N // 4
P4 = NPAD // 4
N8 = N // 8
P8 = NPAD // 8

_MESH = plsc.VectorSubcoreMesh(
    core_axis_name="c", subcore_axis_name="s",
    num_cores=NCORES, num_subcores=NSUB)
_SC_PARAMS = pltpu.CompilerParams(use_tc_tiling_on_sc=False)


@functools.partial(
    pl.kernel,
    out_type=jax.ShapeDtypeStruct((NCORES * NPAD, NC_OUT), jnp.float32),
    mesh=_MESH,
    compiler_params=_SC_PARAMS,
    scratch_types=[
        pltpu.VMEM((NB, BATCH), jnp.int32),
        pltpu.VMEM((BATCH, NC_OUT), jnp.float32),
        pltpu.VMEM_SHARED((NPAD, NC_OUT), jnp.float32),
        pltpu.SemaphoreType.DMA,
    ],
)
def _deg_kernel(dst_hbm, ones_hbm, zeros_hbm, out_hbm, didx, ones_v, acc,
                sem):
    c = lax.axis_index("c")
    s = lax.axis_index("s")
    w = c * NSUB + s
    pltpu.sync_copy(dst_hbm.at[w], didx)
    pltpu.sync_copy(ones_hbm, ones_v)
    base = s * NODES_PER_SUB
    pltpu.sync_copy(zeros_hbm.at[pl.ds(base, NODES_PER_SUB)],
                    acc.at[pl.ds(base, NODES_PER_SUB)])
    plsc.subcore_barrier()

    def fire(i, carry):
        for k in range(DEG_GRP):
            pltpu.async_copy(ones_v, acc.at[didx.at[i * DEG_GRP + k]],
                             sem, add=True)
        return carry

    lax.fori_loop(0, NB // DEG_GRP, fire, 0)

    def drain(i, carry):
        pltpu.make_async_copy(ones_v, acc.at[didx.at[0]], sem).wait()
        return carry

    lax.fori_loop(0, NB, drain, 0)
    plsc.subcore_barrier()
    pltpu.sync_copy(acc.at[pl.ds(base, NODES_PER_SUB)],
                    out_hbm.at[pl.ds(c * NPAD + base, NODES_PER_SUB)])


def _make_conv_scatter(feat):
    @functools.partial(
        pl.kernel,
        out_type=jax.ShapeDtypeStruct((NCORES * NPAD, feat), jnp.float32),
        mesh=_MESH,
        compiler_params=_SC_PARAMS,
        scratch_types=[
            pltpu.VMEM((NB, BATCH), jnp.int32),
            pltpu.VMEM((NB, BATCH), jnp.int32),
            pltpu.VMEM((NSLOT, BATCH, feat), jnp.float32),
            pltpu.VMEM_SHARED((NPAD, feat), jnp.float32),
        ] + [pltpu.SemaphoreType.DMA] * NSLOT,
    )
    def conv_scatter(hs_hbm, edge_hbm, zeros_hbm, out_hbm,
                     sidx, didx, rows, acc, *sems):
        c = lax.axis_index("c")
        s = lax.axis_index("s")
        w = c * NSUB + s
        pltpu.sync_copy(edge_hbm.at[0, w], sidx)
        pltpu.sync_copy(edge_hbm.at[1, w], didx)
        base = s * NODES_PER_SUB
        pltpu.sync_copy(zeros_hbm.at[pl.ds(base, NODES_PER_SUB)],
                        acc.at[pl.ds(base, NODES_PER_SUB)])
        plsc.subcore_barrier()

        for k in range(NSLOT):
            pltpu.async_copy(hs_hbm.at[sidx.at[k]], rows.at[k], sems[k])

        def body(i, carry):
            for k in range(NSLOT):
                b = i * NSLOT + k
                pltpu.make_async_copy(hs_hbm.at[sidx.at[0]], rows.at[k],
                                      sems[k]).wait()
                pltpu.sync_copy(rows.at[k], acc.at[didx.at[b]], add=True)

                @pl.when(b + NSLOT < NB)
                def _():
                    pltpu.async_copy(hs_hbm.at[sidx.at[b + NSLOT]],
                                     rows.at[k], sems[k])
            return carry

        lax.fori_loop(0, NB // NSLOT, body, 0)

        plsc.subcore_barrier()
        pltpu.sync_copy(acc.at[pl.ds(base, NODES_PER_SUB)],
                        out_hbm.at[pl.ds(c * NPAD + base, NODES_PER_SUB)])

    return conv_scatter


_conv32 = _make_conv_scatter(HID)
_conv16 = _make_conv_scatter(NC_OUT)


def _dense1_body(x4_ref, w4_ref, d32_ref, hph_ref, hs_ref):
    dsum = d32_ref[0:P4] + d32_ref[P4:2 * P4] + 1.0
    dinv = lax.rsqrt(dsum)[0:N4]
    hm = jnp.dot(x4_ref[...], w4_ref[...],
                 preferred_element_type=jnp.float32)
    hph_ref[...] = hm
    hs_ref[...] = hm[:, :128] * dinv


def _combine_body(p_ref, hph_ref, d32_ref, d16_ref, w2_ref, b1_ref, bmlp_ref,
                  g2_ref, g2s_ref):
    dsum = d32_ref[0:P4] + d32_ref[P4:2 * P4] + 1.0
    dinv = lax.rsqrt(dsum)[0:N4]
    psum = (p_ref[0:P4] + p_ref[P4:2 * P4])[0:N4]
    out1 = dinv * psum + dinv * dinv * hph_ref[:, :128] + b1_ref[...]
    x2 = jnp.maximum(out1, 0.0) + hph_ref[:, 128:] + bmlp_ref[...]
    g2 = jnp.dot(x2, w2_ref[...],
                 preferred_element_type=jnp.float32)
    dq = d16_ref[0:P4] + d16_ref[P4:2 * P4] + 1.0
    dinvq = lax.rsqrt(dq)[0:N4]
    g2_ref[...] = g2
    g2s_ref[...] = g2 * dinvq


def _final_body(q_ref, g2_ref, d8_ref, b2_ref, s_ref, o_ref):
    d8 = d8_ref[0:P8] + d8_ref[P8:2 * P8] + 1.0
    dinv8 = lax.rsqrt(d8)[0:N8]
    qsum = (q_ref[0:P8] + q_ref[P8:2 * P8])[0:N8]
    out2 = dinv8 * qsum + dinv8 * dinv8 * g2_ref[...] + b2_ref[...]
    e = jnp.exp(out2 - jnp.max(out2))
    denom = jnp.dot(e, s_ref[...], preferred_element_type=jnp.float32)
    o_ref[...] = e / denom


_dense1 = pl.pallas_call(
    _dense1_body,
    out_shape=[jax.ShapeDtypeStruct((N4, 256), jnp.float32),
               jax.ShapeDtypeStruct((N4, 128), jnp.float32)])

_combine = pl.pallas_call(
    _combine_body,
    out_shape=[jax.ShapeDtypeStruct((N4, 64), jnp.float32),
               jax.ShapeDtypeStruct((N4, 64), jnp.float32)])

_final = pl.pallas_call(
    _final_body, out_shape=jax.ShapeDtypeStruct((N8, 128), jnp.float32))


def kernel(x, edge_index, g, A_k, D, Kindices, de, M, I,
           W1, b1, Wmlp, bmlp, W2, b2):
    f32 = jnp.float32
    edge_r = edge_index.reshape(2, NW, NB, BATCH)
    dst_r = edge_index[1].reshape(NW, NB, BATCH)
    ones16 = jnp.ones((BATCH, NC_OUT), f32)
    zeros32 = jnp.zeros((NPAD, HID), f32)
    zeros16 = jnp.zeros((NPAD, NC_OUT), f32)

    x4 = x.reshape(N4, 4 * IN_DIM)
    w4 = jnp.concatenate(
        [_block_diag(W1, W1, W1, W1), _block_diag(Wmlp, Wmlp, Wmlp, Wmlp)],
        axis=1)
    w2bd = _block_diag(W2, W2, W2, W2)
    b1p = jnp.tile(b1, 4).reshape(1, 128)
    bmlpp = jnp.tile(bmlp, 4).reshape(1, 128)
    b2p = jnp.tile(b2, 8).reshape(1, 128)
    lane = jnp.arange(128, dtype=jnp.int32)
    smat = (lane[:, None] // NC_OUT == lane[None, :] // NC_OUT).astype(f32)

    d16 = _deg_kernel(dst_r, ones16, zeros16)
    d32 = jnp.concatenate([d16, d16], axis=1)
    d32_v = d32.reshape(NCORES * P4, 128)
    d16q = d16.reshape(NCORES * P4, 64)
    d8_v = d16.reshape(NCORES * P8, 128)

    hph, hs_p = _dense1(x4, w4, d32_v)
    p = _conv32(hs_p.reshape(N, HID), edge_r, zeros32)
    g2_p4, g2s_p4 = _combine(p.reshape(NCORES * P4, 128), hph, d32_v, d16q,
                             w2bd, b1p, bmlpp)
    g2s_lin = g2s_p4.reshape(N, NC_OUT)
    q = _conv16(g2s_lin, edge_r, zeros16)
    g2_p8 = g2_p4.reshape(N, NC_OUT).reshape(N8, 128)
    out_p8 = _final(q.reshape(NCORES * P8, 128), g2_p8, d8_v, b2p, smat)
    return out_p8.reshape(N, NC_OUT)


# speedup vs baseline: 1.0796x; 1.0796x over previous
import functools

import jax
import jax.numpy as jnp
from jax import lax
from jax.experimental import pallas as pl
from jax.experimental.pallas import tpu as pltpu
from jax.experimental.pallas import tpu_sc as plsc
from jax.scipy.linalg import block_diag as _block_diag

N = 10000
E = 320000
IN_DIM = 128
HID = 32
NC_OUT = 16

NCORES = 2
NSUB = 16
NW = NCORES * NSUB
EPT = E // NW
BATCH = 125
NB = EPT // BATCH
NPAD = 10240
NODES_PER_SUB = NPAD // NSUB
NSLOT = 5
DEG_GRP = 10

N4 = N // 4
P4 = NPAD // 4
N8 = N // 8
P8 = NPAD // 8

_MESH = plsc.VectorSubcoreMesh(
    core_axis_name="c", subcore_axis_name="s",
    num_cores=NCORES, num_subcores=NSUB)
_SC_PARAMS = pltpu.CompilerParams(use_tc_tiling_on_sc=False)


@functools.partial(
    pl.kernel,
    out_type=jax.ShapeDtypeStruct((NCORES * NPAD, HID), jnp.float32),
    mesh=_MESH,
    compiler_params=_SC_PARAMS,
    scratch_types=[
        pltpu.VMEM((NB, BATCH), jnp.int32),
        pltpu.VMEM((BATCH, HID), jnp.float32),
        pltpu.VMEM_SHARED((NPAD, HID), jnp.float32),
        pltpu.SemaphoreType.DMA,
    ],
)
def _deg_kernel(dst_hbm, ones_hbm, zeros_hbm, out_hbm, didx, ones_v, acc,
                sem):
    c = lax.axis_index("c")
    s = lax.axis_index("s")
    w = c * NSUB + s
    pltpu.sync_copy(dst_hbm.at[w], didx)
    pltpu.sync_copy(ones_hbm, ones_v)
    base = s * NODES_PER_SUB
    pltpu.sync_copy(zeros_hbm.at[pl.ds(base, NODES_PER_SUB)],
                    acc.at[pl.ds(base, NODES_PER_SUB)])
    plsc.subcore_barrier()

    def fire(i, carry):
        for k in range(DEG_GRP):
            pltpu.async_copy(ones_v, acc.at[didx.at[i * DEG_GRP + k]],
                             sem, add=True)
        return carry

    lax.fori_loop(0, NB // DEG_GRP, fire, 0)

    def drain(i, carry):
        pltpu.make_async_copy(ones_v, acc.at[didx.at[0]], sem).wait()
        return carry

    lax.fori_loop(0, NB, drain, 0)
    plsc.subcore_barrier()
    pltpu.sync_copy(acc.at[pl.ds(base, NODES_PER_SUB)],
                    out_hbm.at[pl.ds(c * NPAD + base, NODES_PER_SUB)])


def _make_conv_scatter(feat):
    @functools.partial(
        pl.kernel,
        out_type=jax.ShapeDtypeStruct((NCORES * NPAD, feat), jnp.float32),
        mesh=_MESH,
        compiler_params=_SC_PARAMS,
        scratch_types=[
            pltpu.VMEM((NB, BATCH), jnp.int32),
            pltpu.VMEM((NB, BATCH), jnp.int32),
            pltpu.VMEM((NSLOT, BATCH, feat), jnp.float32),
            pltpu.VMEM_SHARED((NPAD, feat), jnp.float32),
        ] + [pltpu.SemaphoreType.DMA] * NSLOT,
    )
    def conv_scatter(hs_hbm, edge_hbm, zeros_hbm, out_hbm,
                     sidx, didx, rows, acc, *sems):
        c = lax.axis_index("c")
        s = lax.axis_index("s")
        w = c * NSUB + s
        pltpu.sync_copy(edge_hbm.at[0, w], sidx)
        pltpu.sync_copy(edge_hbm.at[1, w], didx)
        base = s * NODES_PER_SUB
        pltpu.sync_copy(zeros_hbm.at[pl.ds(base, NODES_PER_SUB)],
                        acc.at[pl.ds(base, NODES_PER_SUB)])
        plsc.subcore_barrier()

        for k in range(NSLOT):
            pltpu.async_copy(hs_hbm.at[sidx.at[k]], rows.at[k], sems[k])

        def body(i, carry):
            for k in range(NSLOT):
                b = i * NSLOT + k
                pltpu.make_async_copy(hs_hbm.at[sidx.at[0]], rows.at[k],
                                      sems[k]).wait()
                pltpu.sync_copy(rows.at[k], acc.at[didx.at[b]], add=True)

                @pl.when(b + NSLOT < NB)
                def _():
                    pltpu.async_copy(hs_hbm.at[sidx.at[b + NSLOT]],
                                     rows.at[k], sems[k])
            return carry

        lax.fori_loop(0, NB // NSLOT, body, 0)

        plsc.subcore_barrier()
        pltpu.sync_copy(acc.at[pl.ds(base, NODES_PER_SUB)],
                        out_hbm.at[pl.ds(c * NPAD + base, NODES_PER_SUB)])

    return conv_scatter


_conv32 = _make_conv_scatter(HID)
_conv16 = _make_conv_scatter(NC_OUT)


def _dense1_body(x4_ref, w4_ref, d32_ref, hph_ref, hs_ref):
    dsum = d32_ref[0:P4] + d32_ref[P4:2 * P4] + 1.0
    dinv = lax.rsqrt(dsum)[0:N4]
    hm = jnp.dot(x4_ref[...], w4_ref[...],
                 preferred_element_type=jnp.float32)
    hph_ref[...] = hm
    hs_ref[...] = hm[:, :128] * dinv


def _combine_body(p_ref, hph_ref, d32_ref, d16_ref, w2_ref, b1_ref, bmlp_ref,
                  g2_ref, g2s_ref):
    dsum = d32_ref[0:P4] + d32_ref[P4:2 * P4] + 1.0
    dinv = lax.rsqrt(dsum)[0:N4]
    psum = (p_ref[0:P4] + p_ref[P4:2 * P4])[0:N4]
    out1 = dinv * psum + dinv * dinv * hph_ref[:, :128] + b1_ref[...]
    x2 = jnp.maximum(out1, 0.0) + hph_ref[:, 128:] + bmlp_ref[...]
    g2 = jnp.dot(x2, w2_ref[...],
                 preferred_element_type=jnp.float32)
    dq = d16_ref[0:P4] + d16_ref[P4:2 * P4] + 1.0
    dinvq = lax.rsqrt(dq)[0:N4]
    g2_ref[...] = g2
    g2s_ref[...] = g2 * dinvq


def _final_body(q_ref, g2_ref, d8_ref, b2_ref, s_ref, o_ref):
    d8 = d8_ref[0:P8] + d8_ref[P8:2 * P8] + 1.0
    dinv8 = lax.rsqrt(d8)[0:N8]
    qsum = (q_ref[0:P8] + q_ref[P8:2 * P8])[0:N8]
    out2 = dinv8 * qsum + dinv8 * dinv8 * g2_ref[...] + b2_ref[...]
    e = jnp.exp(out2 - jnp.max(out2))
    denom = jnp.dot(e, s_ref[...], preferred_element_type=jnp.float32)
    o_ref[...] = e / denom


_dense1 = pl.pallas_call(
    _dense1_body,
    out_shape=[jax.ShapeDtypeStruct((N4, 256), jnp.float32),
               jax.ShapeDtypeStruct((N4, 128), jnp.float32)])

_combine = pl.pallas_call(
    _combine_body,
    out_shape=[jax.ShapeDtypeStruct((N4, 64), jnp.float32),
               jax.ShapeDtypeStruct((N4, 64), jnp.float32)])

_final = pl.pallas_call(
    _final_body, out_shape=jax.ShapeDtypeStruct((N8, 128), jnp.float32))


def kernel(x, edge_index, g, A_k, D, Kindices, de, M, I,
           W1, b1, Wmlp, bmlp, W2, b2):
    f32 = jnp.float32
    edge_r = edge_index.reshape(2, NW, NB, BATCH)
    dst_r = edge_index[1].reshape(NW, NB, BATCH)
    ones32 = jnp.ones((BATCH, HID), f32)
    zeros32 = jnp.zeros((NPAD, HID), f32)
    zeros16 = jnp.zeros((NPAD, NC_OUT), f32)

    x4 = x.reshape(N4, 4 * IN_DIM)
    w4 = jnp.concatenate(
        [_block_diag(W1, W1, W1, W1), _block_diag(Wmlp, Wmlp, Wmlp, Wmlp)],
        axis=1)
    w2bd = _block_diag(W2, W2, W2, W2)
    b1p = jnp.tile(b1, 4).reshape(1, 128)
    bmlpp = jnp.tile(bmlp, 4).reshape(1, 128)
    b2p = jnp.tile(b2, 8).reshape(1, 128)
    lane = jnp.arange(128, dtype=jnp.int32)
    smat = (lane[:, None] // NC_OUT == lane[None, :] // NC_OUT).astype(f32)

    degp = _deg_kernel(dst_r, ones32, zeros32)
    d32_v = degp.reshape(NCORES * P4, 128)
    d16 = degp[:, :NC_OUT]
    d16q = d16.reshape(NCORES * P4, 64)
    d8_v = d16.reshape(NCORES * P8, 128)

    hph, hs_p = _dense1(x4, w4, d32_v)
    p = _conv32(hs_p.reshape(N, HID), edge_r, zeros32)
    g2_p4, g2s_p4 = _combine(p.reshape(NCORES * P4, 128), hph, d32_v, d16q,
                             w2bd, b1p, bmlpp)
    g2s_lin = g2s_p4.reshape(N, NC_OUT)
    q = _conv16(g2s_lin, edge_r, zeros16)
    g2_p8 = g2_p4.reshape(N, NC_OUT).reshape(N8, 128)
    out_p8 = _final(q.reshape(NCORES * P8, 128), g2_p8, d8_v, b2p, smat)
    return out_p8.reshape(N, NC_OUT)


# speedup vs baseline: 1.1418x; 1.0576x over previous
import functools

import jax
import jax.numpy as jnp
from jax import lax
from jax.experimental import pallas as pl
from jax.experimental.pallas import tpu as pltpu
from jax.experimental.pallas import tpu_sc as plsc
from jax.scipy.linalg import block_diag as _block_diag

N = 10000
E = 320000
IN_DIM = 128
HID = 32
NC_OUT = 16

NCORES = 2
NSUB = 16
NW = NCORES * NSUB
EPT = E // NW
BATCH = 125
NB = EPT // BATCH
NPAD = 10240
NODES_PER_SUB = NPAD // NSUB
NSLOT = 5
DEG_GRP = 10

N4 = N // 4
P4 = NPAD // 4
N8 = N // 8
P8 = NPAD // 8

_MESH = plsc.VectorSubcoreMesh(
    core_axis_name="c", subcore_axis_name="s",
    num_cores=NCORES, num_subcores=NSUB)
_SC_PARAMS = pltpu.CompilerParams(use_tc_tiling_on_sc=False)


@functools.partial(
    pl.kernel,
    out_type=jax.ShapeDtypeStruct((NCORES * NPAD, HID), jnp.float32),
    mesh=_MESH,
    compiler_params=_SC_PARAMS,
    scratch_types=[
        pltpu.VMEM((NB, BATCH), jnp.int32),
        pltpu.VMEM((BATCH, HID), jnp.float32),
        pltpu.VMEM_SHARED((NPAD, HID), jnp.float32),
        pltpu.SemaphoreType.DMA,
    ],
)
def _deg_kernel(edge_hbm, ones_hbm, zeros_hbm, out_hbm, didx, ones_v, acc,
                sem):
    c = lax.axis_index("c")
    s = lax.axis_index("s")
    w = c * NSUB + s
    pltpu.sync_copy(edge_hbm.at[1, w], didx)
    pltpu.sync_copy(ones_hbm, ones_v)
    base = s * NODES_PER_SUB
    pltpu.sync_copy(zeros_hbm.at[pl.ds(base, NODES_PER_SUB)],
                    acc.at[pl.ds(base, NODES_PER_SUB)])
    plsc.subcore_barrier()

    def fire(i, carry):
        for k in range(DEG_GRP):
            pltpu.async_copy(ones_v, acc.at[didx.at[i * DEG_GRP + k]],
                             sem, add=True)
        return carry

    lax.fori_loop(0, NB // DEG_GRP, fire, 0)

    def drain(i, carry):
        pltpu.make_async_copy(ones_v, acc.at[didx.at[0]], sem).wait()
        return carry

    lax.fori_loop(0, NB, drain, 0)
    plsc.subcore_barrier()
    pltpu.sync_copy(acc.at[pl.ds(base, NODES_PER_SUB)],
                    out_hbm.at[pl.ds(c * NPAD + base, NODES_PER_SUB)])


def _make_conv_scatter(feat):
    @functools.partial(
        pl.kernel,
        out_type=jax.ShapeDtypeStruct((NCORES * NPAD, feat), jnp.float32),
        mesh=_MESH,
        compiler_params=_SC_PARAMS,
        scratch_types=[
            pltpu.VMEM((NB, BATCH), jnp.int32),
            pltpu.VMEM((NB, BATCH), jnp.int32),
            pltpu.VMEM((NSLOT, BATCH, feat), jnp.float32),
            pltpu.VMEM_SHARED((NPAD, feat), jnp.float32),
        ] + [pltpu.SemaphoreType.DMA] * NSLOT,
    )
    def conv_scatter(hs_hbm, edge_hbm, zeros_hbm, out_hbm,
                     sidx, didx, rows, acc, *sems):
        c = lax.axis_index("c")
        s = lax.axis_index("s")
        w = c * NSUB + s
        pltpu.sync_copy(edge_hbm.at[0, w], sidx)
        pltpu.sync_copy(edge_hbm.at[1, w], didx)
        base = s * NODES_PER_SUB
        pltpu.sync_copy(zeros_hbm.at[pl.ds(base, NODES_PER_SUB)],
                        acc.at[pl.ds(base, NODES_PER_SUB)])
        plsc.subcore_barrier()

        for k in range(NSLOT):
            pltpu.async_copy(hs_hbm.at[sidx.at[k]], rows.at[k], sems[k])

        def body(i, carry):
            for k in range(NSLOT):
                b = i * NSLOT + k
                pltpu.make_async_copy(hs_hbm.at[sidx.at[0]], rows.at[k],
                                      sems[k]).wait()
                pltpu.sync_copy(rows.at[k], acc.at[didx.at[b]], add=True)

                @pl.when(b + NSLOT < NB)
                def _():
                    pltpu.async_copy(hs_hbm.at[sidx.at[b + NSLOT]],
                                     rows.at[k], sems[k])
            return carry

        lax.fori_loop(0, NB // NSLOT, body, 0)

        plsc.subcore_barrier()
        pltpu.sync_copy(acc.at[pl.ds(base, NODES_PER_SUB)],
                        out_hbm.at[pl.ds(c * NPAD + base, NODES_PER_SUB)])

    return conv_scatter


_conv32 = _make_conv_scatter(HID)
_conv16 = _make_conv_scatter(NC_OUT)


def _mm4_body(x4_ref, w4_ref, hph_ref):
    hph_ref[...] = jnp.dot(x4_ref[...], w4_ref[...],
                           preferred_element_type=jnp.float32)


def _scale4_body(hph_ref, d32_ref, hs_ref):
    dsum = d32_ref[0:P4] + d32_ref[P4:2 * P4] + 1.0
    dinv = lax.rsqrt(dsum)[0:N4]
    hs_ref[...] = hph_ref[:, :128] * dinv


def _combine_body(p_ref, hph_ref, d32_ref, d16_ref, w2_ref, b1_ref, bmlp_ref,
                  g2_ref, g2s_ref):
    dsum = d32_ref[0:P4] + d32_ref[P4:2 * P4] + 1.0
    dinv = lax.rsqrt(dsum)[0:N4]
    psum = (p_ref[0:P4] + p_ref[P4:2 * P4])[0:N4]
    out1 = dinv * psum + dinv * dinv * hph_ref[:, :128] + b1_ref[...]
    x2 = jnp.maximum(out1, 0.0) + hph_ref[:, 128:] + bmlp_ref[...]
    g2 = jnp.dot(x2, w2_ref[...],
                 preferred_element_type=jnp.float32)
    dq = d16_ref[0:P4] + d16_ref[P4:2 * P4] + 1.0
    dinvq = lax.rsqrt(dq)[0:N4]
    g2_ref[...] = g2
    g2s_ref[...] = g2 * dinvq


def _final_body(q_ref, g2_ref, d8_ref, b2_ref, s_ref, o_ref):
    d8 = d8_ref[0:P8] + d8_ref[P8:2 * P8] + 1.0
    dinv8 = lax.rsqrt(d8)[0:N8]
    qsum = (q_ref[0:P8] + q_ref[P8:2 * P8])[0:N8]
    out2 = dinv8 * qsum + dinv8 * dinv8 * g2_ref[...] + b2_ref[...]
    e = jnp.exp(out2 - jnp.max(out2))
    denom = jnp.dot(e, s_ref[...], preferred_element_type=jnp.float32)
    o_ref[...] = e / denom


_mm4 = pl.pallas_call(
    _mm4_body, out_shape=jax.ShapeDtypeStruct((N4, 256), jnp.float32))

_scale4 = pl.pallas_call(
    _scale4_body, out_shape=jax.ShapeDtypeStruct((N4, 128), jnp.float32))

_combine = pl.pallas_call(
    _combine_body,
    out_shape=[jax.ShapeDtypeStruct((N4, 64), jnp.float32),
               jax.ShapeDtypeStruct((N4, 64), jnp.float32)])

_final = pl.pallas_call(
    _final_body, out_shape=jax.ShapeDtypeStruct((N8, 128), jnp.float32))


def kernel(x, edge_index, g, A_k, D, Kindices, de, M, I,
           W1, b1, Wmlp, bmlp, W2, b2):
    f32 = jnp.float32
    edge_r = edge_index.reshape(2, NW, NB, BATCH)
    ones32 = jnp.ones((BATCH, HID), f32)
    zeros32 = jnp.zeros((NPAD, HID), f32)
    zeros16 = jnp.zeros((NPAD, NC_OUT), f32)

    x4 = x.reshape(N4, 4 * IN_DIM)
    w4 = jnp.concatenate(
        [_block_diag(W1, W1, W1, W1), _block_diag(Wmlp, Wmlp, Wmlp, Wmlp)],
        axis=1)
    w2bd = _block_diag(W2, W2, W2, W2)
    b1p = jnp.tile(b1, 4).reshape(1, 128)
    bmlpp = jnp.tile(bmlp, 4).reshape(1, 128)
    b2p = jnp.tile(b2, 8).reshape(1, 128)
    lane = jnp.arange(128, dtype=jnp.int32)
    smat = (lane[:, None] // NC_OUT == lane[None, :] // NC_OUT).astype(f32)

    degp = _deg_kernel(edge_r, ones32, zeros32)
    d32_v = degp.reshape(NCORES * P4, 128)
    d16 = degp[:, :NC_OUT]
    d16q = d16.reshape(NCORES * P4, 64)
    d8_v = d16.reshape(NCORES * P8, 128)

    hph = _mm4(x4, w4)
    hs_p = _scale4(hph, d32_v)
    p = _conv32(hs_p.reshape(N, HID), edge_r, zeros32)
    g2_p4, g2s_p4 = _combine(p.reshape(NCORES * P4, 128), hph, d32_v, d16q,
                             w2bd, b1p, bmlpp)
    g2s_lin = g2s_p4.reshape(N, NC_OUT)
    q = _conv16(g2s_lin, edge_r, zeros16)
    g2_p8 = g2_p4.reshape(N, NC_OUT).reshape(N8, 128)
    out_p8 = _final(q.reshape(NCORES * P8, 128), g2_p8, d8_v, b2p, smat)
    return out_p8.reshape(N, NC_OUT)
